# baseline (speedup 1.0000x reference)
import functools

import jax
import jax.numpy as jnp
from jax import lax
from jax.experimental import pallas as pl
from jax.experimental.pallas import tpu as pltpu
from jax.experimental.pallas import tpu_sc as plsc

_NC = 2
_NS = 16
_NW = _NC * _NS


_NB = 8
_H = 2


def _make_gather(S, V, Dm, B):
    b_per_w = B // _NW
    rows = b_per_w // _H
    units = [(j, h) for j in range(S) for h in range(_H)]
    nu = len(units)
    nb = min(_NB, nu)
    mesh = plsc.VectorSubcoreMesh(core_axis_name="c", subcore_axis_name="s")

    scratch = [pltpu.VMEM((b_per_w,), jnp.int32)]
    scratch += [pltpu.VMEM((rows, Dm), jnp.float32) for _ in range(nb)]
    scratch += [pltpu.SemaphoreType.DMA] * (2 * nb)

    @functools.partial(
        pl.kernel,
        mesh=mesh,
        out_type=jax.ShapeDtypeStruct((S, B, Dm), jnp.float32),
        scratch_types=scratch,
    )
    def gather_kernel(table_hbm, idx_hbm, out_hbm, idx_v, *bufs_and_sems):
        bufs = bufs_and_sems[:nb]
        gsem = bufs_and_sems[nb : 2 * nb]
        ssem = bufs_and_sems[2 * nb :]

        wid = lax.axis_index("s") * _NC + lax.axis_index("c")
        base = wid * b_per_w
        pltpu.sync_copy(idx_hbm.at[pl.ds(base, b_per_w)], idx_v)

        def start_gather(u):
            j, h = units[u]
            return pltpu.async_copy(
                table_hbm.at[j].at[idx_v.at[pl.ds(h * rows, rows)]],
                bufs[u % nb],
                gsem[u % nb],
            )

        def start_scatter(u):
            j, h = units[u]
            return pltpu.async_copy(
                bufs[u % nb],
                out_hbm.at[j].at[pl.ds(base + h * rows, rows)],
                ssem[u % nb],
            )

        gd = [None] * nu
        sd = [None] * nu
        for u in range(nb):
            gd[u] = start_gather(u)
        for u in range(nu):
            if u >= 1 and u - 1 + nb < nu:
                sd[u - 1].wait()
                gd[u - 1 + nb] = start_gather(u - 1 + nb)
            gd[u].wait()
            sd[u] = start_scatter(u)
        for u in range(max(0, nu - nb), nu):
            if sd[u] is not None:
                sd[u].wait()

    return gather_kernel


def kernel(indexs, entity_prompts):
    B = indexs.shape[0]
    V, S, Dm = entity_prompts.shape
    table_t = jnp.transpose(entity_prompts, (1, 0, 2))
    out_t = _make_gather(S, V, Dm, B)(table_t, indexs.astype(jnp.int32))
    return jnp.transpose(out_t, (1, 0, 2))


# speedup vs baseline: 1.0095x; 1.0095x over previous
import functools

import jax
import jax.numpy as jnp
from jax import lax
from jax.experimental import pallas as pl
from jax.experimental.pallas import tpu as pltpu
from jax.experimental.pallas import tpu_sc as plsc

_NC = 2
_NS = 16
_NW = _NC * _NS


_NB = 7
_H = 1


def _make_gather(S, V, Dm, B):
    b_per_w = B // _NW
    rows = b_per_w // _H
    units = [(j, h) for j in range(S) for h in range(_H)]
    nu = len(units)
    nb = min(_NB, nu)
    mesh = plsc.VectorSubcoreMesh(core_axis_name="c", subcore_axis_name="s")

    scratch = [pltpu.VMEM((b_per_w,), jnp.int32)]
    scratch += [pltpu.VMEM((rows, Dm), jnp.float32) for _ in range(nb)]
    scratch += [pltpu.SemaphoreType.DMA] * (2 * nb)

    @functools.partial(
        pl.kernel,
        mesh=mesh,
        out_type=jax.ShapeDtypeStruct((S, B, Dm), jnp.float32),
        scratch_types=scratch,
    )
    def gather_kernel(table_hbm, idx_hbm, out_hbm, idx_v, *bufs_and_sems):
        bufs = bufs_and_sems[:nb]
        gsem = bufs_and_sems[nb : 2 * nb]
        ssem = bufs_and_sems[2 * nb :]

        wid = lax.axis_index("s") * _NC + lax.axis_index("c")
        base = wid * b_per_w
        pltpu.sync_copy(idx_hbm.at[pl.ds(base, b_per_w)], idx_v)

        def start_gather(u):
            j, h = units[u]
            return pltpu.async_copy(
                table_hbm.at[j].at[idx_v.at[pl.ds(h * rows, rows)]],
                bufs[u % nb],
                gsem[u % nb],
            )

        def start_scatter(u):
            j, h = units[u]
            return pltpu.async_copy(
                bufs[u % nb],
                out_hbm.at[j].at[pl.ds(base + h * rows, rows)],
                ssem[u % nb],
            )

        gd = [None] * nu
        sd = [None] * nu
        for u in range(nb):
            gd[u] = start_gather(u)
        for u in range(nu):
            if u >= 1 and u - 1 + nb < nu:
                sd[u - 1].wait()
                gd[u - 1 + nb] = start_gather(u - 1 + nb)
            gd[u].wait()
            sd[u] = start_scatter(u)
        for u in range(max(0, nu - nb), nu):
            if sd[u] is not None:
                sd[u].wait()

    return gather_kernel


def kernel(indexs, entity_prompts):
    B = indexs.shape[0]
    V, S, Dm = entity_prompts.shape
    table_t = jnp.transpose(entity_prompts, (1, 0, 2))
    out_t = _make_gather(S, V, Dm, B)(table_t, indexs.astype(jnp.int32))
    return jnp.transpose(out_t, (1, 0, 2))


# speedup vs baseline: 1.0138x; 1.0043x over previous
import functools

import jax
import jax.numpy as jnp
from jax import lax
from jax.experimental import pallas as pl
from jax.experimental.pallas import tpu as pltpu
from jax.experimental.pallas import tpu_sc as plsc

_NC = 2
_NS = 16
_NW = _NC * _NS


_NB = 6
_H = 1


def _make_gather(S, V, Dm, B):
    b_per_w = B // _NW
    rows = b_per_w // _H
    units = [(j, h) for j in range(S) for h in range(_H)]
    nu = len(units)
    nb = min(_NB, nu)
    mesh = plsc.VectorSubcoreMesh(core_axis_name="c", subcore_axis_name="s")

    scratch = [pltpu.VMEM((b_per_w,), jnp.int32)]
    scratch += [pltpu.VMEM((rows, Dm), jnp.float32) for _ in range(nb)]
    scratch += [pltpu.SemaphoreType.DMA] * (2 * nb)

    @functools.partial(
        pl.kernel,
        mesh=mesh,
        out_type=jax.ShapeDtypeStruct((S, B, Dm), jnp.float32),
        scratch_types=scratch,
    )
    def gather_kernel(table_hbm, idx_hbm, out_hbm, idx_v, *bufs_and_sems):
        bufs = bufs_and_sems[:nb]
        gsem = bufs_and_sems[nb : 2 * nb]
        ssem = bufs_and_sems[2 * nb :]

        wid = lax.axis_index("s") * _NC + lax.axis_index("c")
        base = wid * b_per_w
        pltpu.sync_copy(idx_hbm.at[pl.ds(base, b_per_w)], idx_v)

        def start_gather(u):
            j, h = units[u]
            return pltpu.async_copy(
                table_hbm.at[j].at[idx_v.at[pl.ds(h * rows, rows)]],
                bufs[u % nb],
                gsem[u % nb],
            )

        def start_scatter(u):
            j, h = units[u]
            return pltpu.async_copy(
                bufs[u % nb],
                out_hbm.at[j].at[pl.ds(base + h * rows, rows)],
                ssem[u % nb],
            )

        gd = [None] * nu
        sd = [None] * nu
        for u in range(nb):
            gd[u] = start_gather(u)
        for u in range(nu):
            if u >= 1 and u - 1 + nb < nu:
                sd[u - 1].wait()
                gd[u - 1 + nb] = start_gather(u - 1 + nb)
            gd[u].wait()
            sd[u] = start_scatter(u)
        for u in range(max(0, nu - nb), nu):
            if sd[u] is not None:
                sd[u].wait()

    return gather_kernel


def kernel(indexs, entity_prompts):
    B = indexs.shape[0]
    V, S, Dm = entity_prompts.shape
    table_t = jnp.transpose(entity_prompts, (1, 0, 2))
    out_t = _make_gather(S, V, Dm, B)(table_t, indexs.astype(jnp.int32))
    return jnp.transpose(out_t, (1, 0, 2))


# speedup vs baseline: 1.0153x; 1.0015x over previous
import functools

import jax
import jax.numpy as jnp
from jax import lax
from jax.experimental import pallas as pl
from jax.experimental.pallas import tpu as pltpu
from jax.experimental.pallas import tpu_sc as plsc

_NC = 2
_NS = 16
_NW = _NC * _NS


_NB = 6
_H = 1


def _make_gather(S, V, Dm, B):
    b_per_w = B // _NW
    rows = b_per_w // _H
    units = [(j, h) for j in range(S) for h in range(_H)]
    nu = len(units)
    nb = min(_NB, nu)
    mesh = plsc.VectorSubcoreMesh(core_axis_name="c", subcore_axis_name="s")

    scratch = [pltpu.VMEM((b_per_w,), jnp.int32)]
    scratch += [pltpu.VMEM((rows, Dm), jnp.float32) for _ in range(nb)]
    scratch += [pltpu.SemaphoreType.DMA] * (2 * nb)

    @functools.partial(
        pl.kernel,
        mesh=mesh,
        out_type=jax.ShapeDtypeStruct((S, B, Dm), jnp.float32),
        scratch_types=scratch,
        compiler_params=pltpu.CompilerParams(
            skip_device_barrier=True,
            disable_bounds_checks=True,
        ),
    )
    def gather_kernel(table_hbm, idx_hbm, out_hbm, idx_v, *bufs_and_sems):
        bufs = bufs_and_sems[:nb]
        gsem = bufs_and_sems[nb : 2 * nb]
        ssem = bufs_and_sems[2 * nb :]

        wid = lax.axis_index("s") * _NC + lax.axis_index("c")
        base = wid * b_per_w
        pltpu.sync_copy(idx_hbm.at[pl.ds(base, b_per_w)], idx_v)

        def start_gather(u):
            j, h = units[u]
            return pltpu.async_copy(
                table_hbm.at[j].at[idx_v.at[pl.ds(h * rows, rows)]],
                bufs[u % nb],
                gsem[u % nb],
            )

        def start_scatter(u):
            j, h = units[u]
            return pltpu.async_copy(
                bufs[u % nb],
                out_hbm.at[j].at[pl.ds(base + h * rows, rows)],
                ssem[u % nb],
            )

        gd = [None] * nu
        sd = [None] * nu
        for u in range(nb):
            gd[u] = start_gather(u)
        for u in range(nu):
            if u >= 1 and u - 1 + nb < nu:
                sd[u - 1].wait()
                gd[u - 1 + nb] = start_gather(u - 1 + nb)
            gd[u].wait()
            sd[u] = start_scatter(u)
        for u in range(max(0, nu - nb), nu):
            if sd[u] is not None:
                sd[u].wait()

    return gather_kernel


def kernel(indexs, entity_prompts):
    B = indexs.shape[0]
    V, S, Dm = entity_prompts.shape
    table_t = jnp.transpose(entity_prompts, (1, 0, 2))
    out_t = _make_gather(S, V, Dm, B)(table_t, indexs.astype(jnp.int32))
    return jnp.transpose(out_t, (1, 0, 2))
